# baseline (speedup 1.0000x reference)
import functools

import jax
import jax.numpy as jnp
from jax import lax
from jax.experimental import pallas as pl
from jax.experimental.pallas import tpu as pltpu
from jax.experimental.pallas import tpu_sc as plsc

N = 10000
E = 320000
DF = 128
H = 8
HD = 16
NCLS = 7
NEG = 0.2

NP = 10240
C = 128
NTILES = 32
NBUF = 2
CHUNKS = -(-(E // (NTILES * C) + 1) // NBUF) * NBUF
PER_TILE = CHUNKS * C
EPAD = PER_TILE * NTILES
ROWS_PER_TILE = NP // 16

_sc_mesh = plsc.VectorSubcoreMesh(core_axis_name="c", subcore_axis_name="s")
_sc_params = pltpu.CompilerParams(use_tc_tiling_on_sc=False)


def _js(*shape):
    return jax.ShapeDtypeStruct(shape, jnp.float32)


def _k1_body(x_ref, w1_ref, afwd_ref, arev_ref, xh_out, sfwd_out, srev_out, m_out):
    x = x_ref[...]
    xh = jnp.dot(x, w1_ref[...], preferred_element_type=jnp.float32)
    xh_out[...] = xh
    sf = jnp.dot(xh, afwd_ref[...], preferred_element_type=jnp.float32)
    sr = jnp.dot(xh, arev_ref[...], preferred_element_type=jnp.float32)
    sfwd_out[...] = sf
    srev_out[...] = sr
    msum = jnp.max(sf, axis=0) + jnp.max(sr, axis=0)
    m16 = jnp.maximum(msum, NEG * msum)
    m_out[...] = jnp.broadcast_to(m16[None, :], (8, 16))


K3_BR = 2048


def _k3_body(acca_ref, accb_ref, inva_out, dinvw_out, binvw_out):
    br = K3_BR
    pa = acca_ref[0] + acca_ref[1]
    pb = accb_ref[0] + accb_ref[1]
    degn = pa[:, 8:9]
    dege = pb[:, 0:1]
    dinv = jnp.where(degn > 0, 1.0 / degn, 0.0)
    binv = jnp.where(dege > 0, 1.0 / dege, 0.0)
    ia = 1.0 / (pa + 1e-16)
    colid = lax.broadcasted_iota(jnp.int32, (br, 16), 1)
    out = jnp.where(colid < 8, ia, 0.0)
    out = jnp.where(colid == 8, jnp.broadcast_to(dinv, (br, 16)), out)
    out = jnp.where(colid == 9, jnp.broadcast_to(binv, (br, 16)), out)
    inva_out[...] = out
    ia_rep = jnp.broadcast_to(ia[:, :8, None], (br, 8, HD)).reshape(br, DF)
    dinvw_out[...] = jnp.broadcast_to(dinv, (br, DF)) * ia_rep
    binvw_out[...] = jnp.broadcast_to(binv, (br, DF))


def _k5_body(p_ref, binvw_ref, out):
    out[...] = (p_ref[0] + p_ref[1]) * binvw_ref[...]


def _k7_body(p_ref, dinvw_ref, b1_ref, w2p_ref, out):
    o = (p_ref[0] + p_ref[1]) * dinvw_ref[...] + b1_ref[...]
    h = jnp.where(o > 0, o, jnp.exp(o) - 1.0)
    out[...] = jnp.dot(h, w2p_ref[...], preferred_element_type=jnp.float32)


def _k9_body(p_ref, inva_ref, out):
    binv = inva_ref[:, 9:10]
    out[...] = (p_ref[0] + p_ref[1]) * jnp.broadcast_to(binv, (NP, 16))


def _k11_body(p_ref, inva_ref, b2p_ref, out):
    dinv = inva_ref[:, 8:9]
    o = (p_ref[0] + p_ref[1]) * jnp.broadcast_to(dinv, (NP, 16)) + b2p_ref[...]
    colid = lax.broadcasted_iota(jnp.int32, (NP, 16), 1)
    o = jnp.where(colid < NCLS, o, -1e30)
    m = jnp.max(o, axis=1, keepdims=True)
    ex = jnp.where(colid < NCLS, jnp.exp(o - m), 0.0)
    lse = jnp.log(jnp.sum(ex, axis=1, keepdims=True))
    out[...] = o - m - lse


def _zero_rows(buf, nrows, width=16):
    z = jnp.zeros((16,), jnp.float32)

    def body(i, _):
        for k in range(width // 16):
            buf[i, pl.ds(k * 16, 16)] = z
        return 0

    lax.fori_loop(0, nrows, body, 0)


def _k2_body(row_hbm, col_hbm, sfwd_hbm, srev_hbm, mv_hbm,
             a_out, acca_out, accb_out,
             row_v, col_v, g1, g2, a_buf, ones_buf, zbuf, mv_v,
             acca, accb, gsem):
    c = lax.axis_index("c")
    s = lax.axis_index("s")
    wid = c * 16 + s
    _zero_rows(zbuf, ROWS_PER_TILE)
    pltpu.sync_copy(zbuf, acca.at[pl.ds(s * ROWS_PER_TILE, ROWS_PER_TILE), :])
    pltpu.sync_copy(zbuf, accb.at[pl.ds(s * ROWS_PER_TILE, ROWS_PER_TILE), :])
    pltpu.sync_copy(mv_hbm, mv_v)
    idx16 = lax.iota(jnp.int32, 16)
    onerow = jnp.where(idx16 == 0, 1.0, 0.0).astype(jnp.float32)

    def initones(i, _):
        ones_buf[i, :] = onerow
        return 0

    lax.fori_loop(0, C, initones, 0)
    mv = mv_v[...]
    plsc.subcore_barrier()

    base0 = wid * PER_TILE

    def outer(i, _):
        base = base0 + i * (NBUF * C)
        for b in range(NBUF):
            pltpu.sync_copy(row_hbm.at[pl.ds(base + b * C, C)], row_v[b])
            pltpu.sync_copy(col_hbm.at[pl.ds(base + b * C, C)], col_v[b])
        gcp = [(pltpu.async_copy(sfwd_hbm.at[row_v[b]], g1[b], gsem[b]),
                pltpu.async_copy(srev_hbm.at[col_v[b]], g2[b], gsem[b]))
               for b in range(NBUF)]
        for b in range(NBUF):
            gcp[b][0].wait()
            gcp[b][1].wait()

            def edge(e, _):
                z = g1[b][e, :] + g2[b][e, :]
                zl = jnp.maximum(z, NEG * z)
                a_buf[b][e, :] = jnp.exp(zl - mv)
                return 0

            lax.fori_loop(0, C, edge, 0, unroll=2)
            pltpu.sync_copy(a_buf[b], a_out.at[pl.ds(base + b * C, C), :])
            pltpu.sync_copy(a_buf[b], acca.at[row_v[b]], add=True)
            pltpu.sync_copy(ones_buf, accb.at[col_v[b]], add=True)
        return 0

    lax.fori_loop(0, CHUNKS // NBUF, outer, 0)
    plsc.subcore_barrier()
    sl = pl.ds(s * ROWS_PER_TILE, ROWS_PER_TILE)
    pltpu.sync_copy(acca.at[sl, :], acca_out.at[c, sl, :])
    pltpu.sync_copy(accb.at[sl, :], accb_out.at[c, sl, :])


def _kprop_body(with_ia, gidx_hbm, sidx_hbm, a_hbm, inva_hbm, tab_hbm,
                out_p,
                gi_v, si_v, ia, a_buf, w_buf, rows,
                acc, gsem):
    c = lax.axis_index("c")
    s = lax.axis_index("s")
    wid = c * 16 + s
    _zero_rows(rows[0], C, DF)
    for off in range(0, ROWS_PER_TILE, C):
        n = min(C, ROWS_PER_TILE - off)
        pltpu.sync_copy(rows[0].at[pl.ds(0, n), :],
                        acc.at[pl.ds(s * ROWS_PER_TILE + off, n), :])
    plsc.subcore_barrier()

    base0 = wid * PER_TILE

    def outer(i, _):
        base = base0 + i * (NBUF * C)
        gcp = []
        for b in range(NBUF):
            pltpu.sync_copy(gidx_hbm.at[pl.ds(base + b * C, C)], gi_v[b])
            pltpu.sync_copy(sidx_hbm.at[pl.ds(base + b * C, C)], si_v[b])
        for b in range(NBUF):
            cps = [pltpu.async_copy(tab_hbm.at[gi_v[b]], rows[b], gsem[b]),
                   pltpu.async_copy(a_hbm.at[pl.ds(base + b * C, C), :],
                                    a_buf[b], gsem[b])]
            if with_ia:
                cps.append(pltpu.async_copy(inva_hbm.at[gi_v[b]], ia[b], gsem[b]))
            gcp.append(cps)
        scp = []
        for b in range(NBUF):
            for cp in gcp[b]:
                cp.wait()
            if with_ia:

                def wcomp(e, _):
                    w_buf[e, :] = a_buf[b][e, :] * ia[b][e, :]
                    return 0

                lax.fori_loop(0, C, wcomp, 0, unroll=4)
            wsrc = w_buf if with_ia else a_buf[b]

            def edge(e, _):
                wrow = wsrc[e, :]
                for hh in range(H):
                    sl = pl.ds(hh * HD, HD)
                    wv = jnp.full((HD,), wrow[hh], jnp.float32)
                    rows[b][e, sl] = rows[b][e, sl] * wv
                return 0

            lax.fori_loop(0, C, edge, 0, unroll=2)
            pltpu.sync_copy(rows[b], acc.at[si_v[b]], add=True)
        return 0

    lax.fori_loop(0, CHUNKS // NBUF, outer, 0)
    plsc.subcore_barrier()
    sl = pl.ds(s * ROWS_PER_TILE, ROWS_PER_TILE)
    pltpu.sync_copy(acc.at[sl, :], out_p.at[c, sl, :])


def _kthin_body(gidx_hbm, sidx_hbm, tab_hbm,
                out_p,
                gi_v, si_v, g, zbuf,
                acc, gsem):
    c = lax.axis_index("c")
    s = lax.axis_index("s")
    wid = c * 16 + s
    _zero_rows(zbuf, ROWS_PER_TILE)
    pltpu.sync_copy(zbuf, acc.at[pl.ds(s * ROWS_PER_TILE, ROWS_PER_TILE), :])
    plsc.subcore_barrier()

    base0 = wid * PER_TILE

    def outer(i, _):
        base = base0 + i * (NBUF * C)
        for b in range(NBUF):
            pltpu.sync_copy(gidx_hbm.at[pl.ds(base + b * C, C)], gi_v[b])
            pltpu.sync_copy(sidx_hbm.at[pl.ds(base + b * C, C)], si_v[b])
        gcp = [pltpu.async_copy(tab_hbm.at[gi_v[b]], g[b], gsem[b])
               for b in range(NBUF)]
        for b in range(NBUF):
            gcp[b].wait()
            pltpu.sync_copy(g[b], acc.at[si_v[b]], add=True)
        return 0

    lax.fori_loop(0, CHUNKS // NBUF, outer, 0)
    plsc.subcore_barrier()
    sl = pl.ds(s * ROWS_PER_TILE, ROWS_PER_TILE)
    pltpu.sync_copy(acc.at[sl, :], out_p.at[c, sl, :])


def kernel(x, edge_index, W1, att1, b1, W2, b2):
    f32 = jnp.float32
    x_p = jnp.zeros((NP, DF), f32).at[:N].set(x)
    pad = jnp.full((EPAD - E,), N, jnp.int32)
    row_p = jnp.concatenate([edge_index[0], pad])
    col_p = jnp.concatenate([edge_index[1], pad])
    af = att1[0, :, :HD]
    ar = att1[0, :, HD:]
    eye = jnp.eye(H, dtype=f32)
    afwd = (af[:, :, None] * eye[:, None, :]).reshape(DF, H)
    arev = (ar[:, :, None] * eye[:, None, :]).reshape(DF, H)
    afwd = jnp.pad(afwd, ((0, 0), (0, 8)))
    arev = jnp.pad(arev, ((0, 0), (0, 8)))
    w2p = jnp.pad(W2, ((0, 0), (0, 16 - NCLS)))
    b1r = b1.reshape(1, DF)
    b2p = jnp.pad(b2, (0, 16 - NCLS)).reshape(1, 16)

    xh_p, sfwd, srev, m8 = pl.pallas_call(
        _k1_body,
        out_shape=[_js(NP, DF), _js(NP, 16), _js(NP, 16), _js(8, 16)],
    )(x_p, W1, afwd, arev)
    mvec = m8[0]

    k2 = functools.partial(
        pl.kernel,
        mesh=_sc_mesh,
        compiler_params=_sc_params,
        out_type=[_js(EPAD, 16), _js(2, NP, 16), _js(2, NP, 16)],
        scratch_types=[
            tuple(pltpu.VMEM((C,), jnp.int32) for _ in range(NBUF)),
            tuple(pltpu.VMEM((C,), jnp.int32) for _ in range(NBUF)),
            tuple(pltpu.VMEM((C, 16), f32) for _ in range(NBUF)),
            tuple(pltpu.VMEM((C, 16), f32) for _ in range(NBUF)),
            tuple(pltpu.VMEM((C, 16), f32) for _ in range(NBUF)),
            pltpu.VMEM((C, 16), f32),
            pltpu.VMEM((ROWS_PER_TILE, 16), f32),
            pltpu.VMEM((16,), f32),
            pltpu.VMEM_SHARED((NP, 16), f32),
            pltpu.VMEM_SHARED((NP, 16), f32),
            tuple(pltpu.SemaphoreType.DMA for _ in range(NBUF)),
        ],
    )(_k2_body)
    a_e, acca_p, accb_p = k2(row_p, col_p, sfwd, srev, mvec)

    inva, dinvw, binvw = pl.pallas_call(
        _k3_body,
        grid=(NP // K3_BR,),
        in_specs=[
            pl.BlockSpec((2, K3_BR, 16), lambda i: (0, i, 0)),
            pl.BlockSpec((2, K3_BR, 16), lambda i: (0, i, 0)),
        ],
        out_specs=[
            pl.BlockSpec((K3_BR, 16), lambda i: (i, 0)),
            pl.BlockSpec((K3_BR, DF), lambda i: (i, 0)),
            pl.BlockSpec((K3_BR, DF), lambda i: (i, 0)),
        ],
        out_shape=[_js(NP, 16), _js(NP, DF), _js(NP, DF)],
    )(acca_p, accb_p)

    def _mk_kprop(with_ia):
        return functools.partial(
            pl.kernel,
            mesh=_sc_mesh,
            compiler_params=_sc_params,
            out_type=[_js(2, NP, DF)],
            scratch_types=[
                tuple(pltpu.VMEM((C,), jnp.int32) for _ in range(NBUF)),
                tuple(pltpu.VMEM((C,), jnp.int32) for _ in range(NBUF)),
                tuple(pltpu.VMEM((C, 16), f32) for _ in range(NBUF)),
                tuple(pltpu.VMEM((C, 16), f32) for _ in range(NBUF)),
                pltpu.VMEM((C, 16), f32),
                tuple(pltpu.VMEM((C, DF), f32) for _ in range(NBUF)),
                pltpu.VMEM_SHARED((NP, DF), f32),
                tuple(pltpu.SemaphoreType.DMA for _ in range(NBUF)),
            ],
        )(functools.partial(_kprop_body, with_ia))

    (oute_p,) = _mk_kprop(True)(row_p, col_p, a_e, inva, xh_p)

    oute_b = pl.pallas_call(_k5_body, out_shape=_js(NP, DF))(oute_p, binvw)

    (out1_p,) = _mk_kprop(False)(col_p, row_p, a_e, inva, oute_b)

    x2p = pl.pallas_call(_k7_body, out_shape=_js(NP, 16))(out1_p, dinvw, b1r, w2p)

    kthin = functools.partial(
        pl.kernel,
        mesh=_sc_mesh,
        compiler_params=_sc_params,
        out_type=[_js(2, NP, 16)],
        scratch_types=[
            tuple(pltpu.VMEM((C,), jnp.int32) for _ in range(NBUF)),
            tuple(pltpu.VMEM((C,), jnp.int32) for _ in range(NBUF)),
            tuple(pltpu.VMEM((C, 16), f32) for _ in range(NBUF)),
            pltpu.VMEM((ROWS_PER_TILE, 16), f32),
            pltpu.VMEM_SHARED((NP, 16), f32),
            tuple(pltpu.SemaphoreType.DMA for _ in range(NBUF)),
        ],
    )(_kthin_body)
    (acc3_p,) = kthin(row_p, col_p, x2p)

    oute2_b = pl.pallas_call(_k9_body, out_shape=_js(NP, 16))(acc3_p, inva)

    (acc4_p,) = kthin(col_p, row_p, oute2_b)

    logp = pl.pallas_call(_k11_body, out_shape=_js(NP, 16))(acc4_p, inva, b2p)
    return logp[:N, :NCLS]


# speedup vs baseline: 1.2636x; 1.2636x over previous
import functools

import jax
import jax.numpy as jnp
from jax import lax
from jax.experimental import pallas as pl
from jax.experimental.pallas import tpu as pltpu
from jax.experimental.pallas import tpu_sc as plsc

N = 10000
E = 320000
DF = 128
H = 8
HD = 16
NCLS = 7
NEG = 0.2

NP = 10240
C = 128
NTILES = 32
CHUNKS = E // (NTILES * C) + 1
PER_TILE = CHUNKS * C
EPAD = PER_TILE * NTILES
ROWS_PER_TILE = NP // 16

_sc_mesh = plsc.VectorSubcoreMesh(core_axis_name="c", subcore_axis_name="s")
_sc_params = pltpu.CompilerParams(use_tc_tiling_on_sc=False)


def _js(*shape):
    return jax.ShapeDtypeStruct(shape, jnp.float32)


def _k1_body(x_ref, w1_ref, afwd_ref, arev_ref, xh_out, sfwd_out, srev_out, m_out):
    x = x_ref[...]
    xh = jnp.dot(x, w1_ref[...], preferred_element_type=jnp.float32)
    xh_out[...] = xh
    sf = jnp.dot(xh, afwd_ref[...], preferred_element_type=jnp.float32)
    sr = jnp.dot(xh, arev_ref[...], preferred_element_type=jnp.float32)
    sfwd_out[...] = sf
    srev_out[...] = sr
    msum = jnp.max(sf, axis=0) + jnp.max(sr, axis=0)
    m16 = jnp.maximum(msum, NEG * msum)
    m_out[...] = jnp.broadcast_to(m16[None, :], (8, 16))


K3_BR = 2048


def _k3_body(acca_ref, accb_ref, inva_out, dinvw_out, binvw_out):
    br = K3_BR
    pa = acca_ref[0] + acca_ref[1]
    pb = accb_ref[0] + accb_ref[1]
    degn = pa[:, 8:9]
    dege = pb[:, 0:1]
    dinv = jnp.where(degn > 0, 1.0 / degn, 0.0)
    binv = jnp.where(dege > 0, 1.0 / dege, 0.0)
    ia = 1.0 / (pa + 1e-16)
    colid = lax.broadcasted_iota(jnp.int32, (br, 16), 1)
    out = jnp.where(colid < 8, ia, 0.0)
    out = jnp.where(colid == 8, jnp.broadcast_to(dinv, (br, 16)), out)
    out = jnp.where(colid == 9, jnp.broadcast_to(binv, (br, 16)), out)
    inva_out[...] = out
    ia_rep = jnp.broadcast_to(ia[:, :8, None], (br, 8, HD)).reshape(br, DF)
    dinvw_out[...] = jnp.broadcast_to(dinv, (br, DF)) * ia_rep
    binvw_out[...] = jnp.broadcast_to(binv, (br, DF))


def _k5_body(p_ref, binvw_ref, out):
    out[...] = (p_ref[0] + p_ref[1]) * binvw_ref[...]


def _k7_body(p_ref, dinvw_ref, b1_ref, w2p_ref, out):
    o = (p_ref[0] + p_ref[1]) * dinvw_ref[...] + b1_ref[...]
    h = jnp.where(o > 0, o, jnp.exp(o) - 1.0)
    out[...] = jnp.dot(h, w2p_ref[...], preferred_element_type=jnp.float32)


def _k9_body(p_ref, inva_ref, out):
    binv = inva_ref[:, 9:10]
    out[...] = (p_ref[0] + p_ref[1]) * jnp.broadcast_to(binv, (NP, 16))


def _k11_body(p_ref, inva_ref, b2p_ref, out):
    dinv = inva_ref[:, 8:9]
    o = (p_ref[0] + p_ref[1]) * jnp.broadcast_to(dinv, (NP, 16)) + b2p_ref[...]
    colid = lax.broadcasted_iota(jnp.int32, (NP, 16), 1)
    o = jnp.where(colid < NCLS, o, -1e30)
    m = jnp.max(o, axis=1, keepdims=True)
    ex = jnp.where(colid < NCLS, jnp.exp(o - m), 0.0)
    lse = jnp.log(jnp.sum(ex, axis=1, keepdims=True))
    out[...] = o - m - lse


def _zero_rows(buf, nrows, width=16):
    z = jnp.zeros((16,), jnp.float32)

    def body(i, _):
        for k in range(width // 16):
            buf[i, pl.ds(k * 16, 16)] = z
        return 0

    lax.fori_loop(0, nrows, body, 0)


def _k2_body(row_hbm, col_hbm, sfwd_hbm, srev_hbm, mv_hbm,
             a_out, acca_out, accb_out,
             row_v, col_v, g1, g2, a_buf, ones_buf, zbuf, mv_v,
             acca, accb, gsem):
    c = lax.axis_index("c")
    s = lax.axis_index("s")
    wid = c * 16 + s
    _zero_rows(zbuf, ROWS_PER_TILE)
    pltpu.sync_copy(zbuf, acca.at[pl.ds(s * ROWS_PER_TILE, ROWS_PER_TILE), :])
    pltpu.sync_copy(zbuf, accb.at[pl.ds(s * ROWS_PER_TILE, ROWS_PER_TILE), :])
    pltpu.sync_copy(mv_hbm, mv_v)
    idx16 = lax.iota(jnp.int32, 16)
    onerow = jnp.where(idx16 == 0, 1.0, 0.0).astype(jnp.float32)

    def initones(i, _):
        ones_buf[i, :] = onerow
        return 0

    lax.fori_loop(0, C, initones, 0)
    mv = mv_v[...]
    plsc.subcore_barrier()

    base0 = wid * PER_TILE

    def chunk(j, _):
        base = base0 + j * C
        pltpu.sync_copy(row_hbm.at[pl.ds(base, C)], row_v)
        pltpu.sync_copy(col_hbm.at[pl.ds(base, C)], col_v)
        cp1 = pltpu.async_copy(sfwd_hbm.at[row_v], g1, gsem)
        cp2 = pltpu.async_copy(srev_hbm.at[col_v], g2, gsem)
        cp1.wait()
        cp2.wait()

        def edge(e, _):
            z = g1[e, :] + g2[e, :]
            zl = jnp.maximum(z, NEG * z)
            a_buf[e, :] = jnp.exp(zl - mv)
            return 0

        lax.fori_loop(0, C, edge, 0, unroll=2)
        pltpu.sync_copy(a_buf, a_out.at[pl.ds(base, C), :])
        pltpu.sync_copy(a_buf, acca.at[row_v], add=True)
        pltpu.sync_copy(ones_buf, accb.at[col_v], add=True)
        return 0

    lax.fori_loop(0, CHUNKS, chunk, 0)
    plsc.subcore_barrier()
    sl = pl.ds(s * ROWS_PER_TILE, ROWS_PER_TILE)
    pltpu.sync_copy(acca.at[sl, :], acca_out.at[c, sl, :])
    pltpu.sync_copy(accb.at[sl, :], accb_out.at[c, sl, :])


def _kprop_body(with_ia, gidx_hbm, sidx_hbm, a_hbm, inva_hbm, tab_hbm,
                out_p,
                gi_v, si_v, ia, a_buf, w_buf, rows,
                acc, gsem):
    c = lax.axis_index("c")
    s = lax.axis_index("s")
    wid = c * 16 + s
    _zero_rows(rows, C, DF)
    for off in range(0, ROWS_PER_TILE, C):
        n = min(C, ROWS_PER_TILE - off)
        pltpu.sync_copy(rows.at[pl.ds(0, n), :],
                        acc.at[pl.ds(s * ROWS_PER_TILE + off, n), :])
    plsc.subcore_barrier()

    base0 = wid * PER_TILE

    def chunk(j, _):
        base = base0 + j * C
        pltpu.sync_copy(gidx_hbm.at[pl.ds(base, C)], gi_v)
        pltpu.sync_copy(sidx_hbm.at[pl.ds(base, C)], si_v)
        cp1 = pltpu.async_copy(tab_hbm.at[gi_v], rows, gsem)
        cp2 = pltpu.async_copy(inva_hbm.at[gi_v], ia, gsem) if with_ia else None
        pltpu.sync_copy(a_hbm.at[pl.ds(base, C), :], a_buf)
        cp1.wait()
        if with_ia:
            cp2.wait()

            def wcomp(e, _):
                w_buf[e, :] = a_buf[e, :] * ia[e, :]
                return 0

            lax.fori_loop(0, C, wcomp, 0, unroll=4)
        wsrc = w_buf if with_ia else a_buf

        def edge(e, _):
            wrow = wsrc[e, :]
            for hh in range(H):
                sl = pl.ds(hh * HD, HD)
                wv = jnp.full((HD,), wrow[hh], jnp.float32)
                rows[e, sl] = rows[e, sl] * wv
            return 0

        lax.fori_loop(0, C, edge, 0, unroll=2)
        pltpu.sync_copy(rows, acc.at[si_v], add=True)
        return 0

    lax.fori_loop(0, CHUNKS, chunk, 0)
    plsc.subcore_barrier()
    sl = pl.ds(s * ROWS_PER_TILE, ROWS_PER_TILE)
    pltpu.sync_copy(acc.at[sl, :], out_p.at[c, sl, :])


def _kthin_body(gidx_hbm, sidx_hbm, tab_hbm,
                out_p,
                gi_v, si_v, g, zbuf,
                acc, gsem):
    c = lax.axis_index("c")
    s = lax.axis_index("s")
    wid = c * 16 + s
    _zero_rows(zbuf, ROWS_PER_TILE)
    pltpu.sync_copy(zbuf, acc.at[pl.ds(s * ROWS_PER_TILE, ROWS_PER_TILE), :])
    plsc.subcore_barrier()

    base0 = wid * PER_TILE

    def chunk(j, _):
        base = base0 + j * C
        pltpu.sync_copy(gidx_hbm.at[pl.ds(base, C)], gi_v)
        pltpu.sync_copy(sidx_hbm.at[pl.ds(base, C)], si_v)
        pltpu.async_copy(tab_hbm.at[gi_v], g, gsem).wait()
        pltpu.sync_copy(g, acc.at[si_v], add=True)
        return 0

    lax.fori_loop(0, CHUNKS, chunk, 0)
    plsc.subcore_barrier()
    sl = pl.ds(s * ROWS_PER_TILE, ROWS_PER_TILE)
    pltpu.sync_copy(acc.at[sl, :], out_p.at[c, sl, :])


def kernel(x, edge_index, W1, att1, b1, W2, b2):
    f32 = jnp.float32
    x_p = jnp.zeros((NP, DF), f32).at[:N].set(x)
    pad = jnp.full((EPAD - E,), N, jnp.int32)
    row_p = jnp.concatenate([edge_index[0], pad])
    col_p = jnp.concatenate([edge_index[1], pad])
    af = att1[0, :, :HD]
    ar = att1[0, :, HD:]
    eye = jnp.eye(H, dtype=f32)
    afwd = (af[:, :, None] * eye[:, None, :]).reshape(DF, H)
    arev = (ar[:, :, None] * eye[:, None, :]).reshape(DF, H)
    afwd = jnp.pad(afwd, ((0, 0), (0, 8)))
    arev = jnp.pad(arev, ((0, 0), (0, 8)))
    w2p = jnp.pad(W2, ((0, 0), (0, 16 - NCLS)))
    b1r = b1.reshape(1, DF)
    b2p = jnp.pad(b2, (0, 16 - NCLS)).reshape(1, 16)

    xh_p, sfwd, srev, m8 = pl.pallas_call(
        _k1_body,
        out_shape=[_js(NP, DF), _js(NP, 16), _js(NP, 16), _js(8, 16)],
    )(x_p, W1, afwd, arev)
    mvec = m8[0]

    k2 = functools.partial(
        pl.kernel,
        mesh=_sc_mesh,
        compiler_params=_sc_params,
        out_type=[_js(EPAD, 16), _js(2, NP, 16), _js(2, NP, 16)],
        scratch_types=[
            pltpu.VMEM((C,), jnp.int32),
            pltpu.VMEM((C,), jnp.int32),
            pltpu.VMEM((C, 16), f32),
            pltpu.VMEM((C, 16), f32),
            pltpu.VMEM((C, 16), f32),
            pltpu.VMEM((C, 16), f32),
            pltpu.VMEM((ROWS_PER_TILE, 16), f32),
            pltpu.VMEM((16,), f32),
            pltpu.VMEM_SHARED((NP, 16), f32),
            pltpu.VMEM_SHARED((NP, 16), f32),
            pltpu.SemaphoreType.DMA,
        ],
    )(_k2_body)
    a_e, acca_p, accb_p = k2(row_p, col_p, sfwd, srev, mvec)

    inva, dinvw, binvw = pl.pallas_call(
        _k3_body,
        grid=(NP // K3_BR,),
        in_specs=[
            pl.BlockSpec((2, K3_BR, 16), lambda i: (0, i, 0)),
            pl.BlockSpec((2, K3_BR, 16), lambda i: (0, i, 0)),
        ],
        out_specs=[
            pl.BlockSpec((K3_BR, 16), lambda i: (i, 0)),
            pl.BlockSpec((K3_BR, DF), lambda i: (i, 0)),
            pl.BlockSpec((K3_BR, DF), lambda i: (i, 0)),
        ],
        out_shape=[_js(NP, 16), _js(NP, DF), _js(NP, DF)],
    )(acca_p, accb_p)

    def _mk_kprop(with_ia):
        return functools.partial(
            pl.kernel,
            mesh=_sc_mesh,
            compiler_params=_sc_params,
            out_type=[_js(2, NP, DF)],
            scratch_types=[
                pltpu.VMEM((C,), jnp.int32),
                pltpu.VMEM((C,), jnp.int32),
                pltpu.VMEM((C, 16), f32),
                pltpu.VMEM((C, 16), f32),
                pltpu.VMEM((C, 16), f32),
                pltpu.VMEM((C, DF), f32),
                pltpu.VMEM_SHARED((NP, DF), f32),
                pltpu.SemaphoreType.DMA,
            ],
        )(functools.partial(_kprop_body, with_ia))

    (oute_p,) = _mk_kprop(True)(row_p, col_p, a_e, inva, xh_p)

    oute_b = pl.pallas_call(_k5_body, out_shape=_js(NP, DF))(oute_p, binvw)

    (out1_p,) = _mk_kprop(False)(col_p, row_p, a_e, inva, oute_b)

    x2p = pl.pallas_call(_k7_body, out_shape=_js(NP, 16))(out1_p, dinvw, b1r, w2p)

    kthin = functools.partial(
        pl.kernel,
        mesh=_sc_mesh,
        compiler_params=_sc_params,
        out_type=[_js(2, NP, 16)],
        scratch_types=[
            pltpu.VMEM((C,), jnp.int32),
            pltpu.VMEM((C,), jnp.int32),
            pltpu.VMEM((C, 16), f32),
            pltpu.VMEM((ROWS_PER_TILE, 16), f32),
            pltpu.VMEM_SHARED((NP, 16), f32),
            pltpu.SemaphoreType.DMA,
        ],
    )(_kthin_body)
    (acc3_p,) = kthin(row_p, col_p, x2p)

    oute2_b = pl.pallas_call(_k9_body, out_shape=_js(NP, 16))(acc3_p, inva)

    (acc4_p,) = kthin(col_p, row_p, oute2_b)

    logp = pl.pallas_call(_k11_body, out_shape=_js(NP, 16))(acc4_p, inva, b2p)
    return logp[:N, :NCLS]


# speedup vs baseline: 1.3608x; 1.0770x over previous
import functools

import jax
import jax.numpy as jnp
from jax import lax
from jax.experimental import pallas as pl
from jax.experimental.pallas import tpu as pltpu
from jax.experimental.pallas import tpu_sc as plsc

N = 10000
E = 320000
DF = 128
H = 8
HD = 16
NCLS = 7
NEG = 0.2

NP = 10240
C = 128
NTILES = 32
CHUNKS = E // (NTILES * C) + 1
PER_TILE = CHUNKS * C
EPAD = PER_TILE * NTILES
ROWS_PER_TILE = NP // 16

_sc_mesh = plsc.VectorSubcoreMesh(core_axis_name="c", subcore_axis_name="s")
_sc_params = pltpu.CompilerParams(use_tc_tiling_on_sc=False)


def _js(*shape):
    return jax.ShapeDtypeStruct(shape, jnp.float32)


def _k1_body(x_ref, w1_ref, afwd_ref, arev_ref, xh_out, sfwd_out, srev_out, m_out):
    x = x_ref[...]
    xh = jnp.dot(x, w1_ref[...], preferred_element_type=jnp.float32)
    xh_out[...] = xh
    sf = jnp.dot(xh, afwd_ref[...], preferred_element_type=jnp.float32)
    sr = jnp.dot(xh, arev_ref[...], preferred_element_type=jnp.float32)
    sfwd_out[...] = sf
    srev_out[...] = sr
    msum = jnp.max(sf, axis=0) + jnp.max(sr, axis=0)
    m16 = jnp.maximum(msum, NEG * msum)
    m_out[...] = jnp.broadcast_to(m16[None, :], (8, 16))


K3_BR = 2048


def _k3_body(acca_ref, accb_ref, inva_out, dinvw_out, binvw_out):
    br = K3_BR
    pa = acca_ref[0] + acca_ref[1]
    pb = accb_ref[0] + accb_ref[1]
    degn = pa[:, 8:9]
    dege = pb[:, 0:1]
    dinv = jnp.where(degn > 0, 1.0 / degn, 0.0)
    binv = jnp.where(dege > 0, 1.0 / dege, 0.0)
    ia = 1.0 / (pa + 1e-16)
    colid = lax.broadcasted_iota(jnp.int32, (br, 16), 1)
    out = jnp.where(colid < 8, ia, 0.0)
    out = jnp.where(colid == 8, jnp.broadcast_to(dinv, (br, 16)), out)
    out = jnp.where(colid == 9, jnp.broadcast_to(binv, (br, 16)), out)
    inva_out[...] = out
    ia_rep = jnp.broadcast_to(ia[:, :8, None], (br, 8, HD)).reshape(br, DF)
    dinvw_out[...] = jnp.broadcast_to(dinv, (br, DF)) * ia_rep
    binvw_out[...] = jnp.broadcast_to(binv, (br, DF))


def _k5_body(p_ref, binvw_ref, out):
    out[...] = (p_ref[0] + p_ref[1]) * binvw_ref[...]


def _k7_body(p_ref, dinvw_ref, b1_ref, w2p_ref, out):
    o = (p_ref[0] + p_ref[1]) * dinvw_ref[...] + b1_ref[...]
    h = jnp.where(o > 0, o, jnp.exp(o) - 1.0)
    out[...] = jnp.dot(h, w2p_ref[...], preferred_element_type=jnp.float32)


def _k9_body(p_ref, inva_ref, out):
    binv = inva_ref[:, 9:10]
    out[...] = (p_ref[0] + p_ref[1]) * jnp.broadcast_to(binv, (NP, 16))


def _k11_body(p_ref, inva_ref, b2p_ref, out):
    dinv = inva_ref[:, 8:9]
    o = (p_ref[0] + p_ref[1]) * jnp.broadcast_to(dinv, (NP, 16)) + b2p_ref[...]
    colid = lax.broadcasted_iota(jnp.int32, (NP, 16), 1)
    o = jnp.where(colid < NCLS, o, -1e30)
    m = jnp.max(o, axis=1, keepdims=True)
    ex = jnp.where(colid < NCLS, jnp.exp(o - m), 0.0)
    lse = jnp.log(jnp.sum(ex, axis=1, keepdims=True))
    out[...] = o - m - lse


def _zero_rows(buf, nrows, width=16):
    z = jnp.zeros((16,), jnp.float32)

    def body(i, _):
        for k in range(width // 16):
            buf[i, pl.ds(k * 16, 16)] = z
        return 0

    lax.fori_loop(0, nrows, body, 0)


def _k2_body(row_hbm, col_hbm, sfwd_hbm, srev_hbm, mv_hbm,
             a_out, acca_out, accb_out,
             row_v, col_v, g1, g2, a_buf, ones_buf, zbuf, mv_v,
             acca, accb, gsem):
    c = lax.axis_index("c")
    s = lax.axis_index("s")
    wid = c * 16 + s
    _zero_rows(zbuf, ROWS_PER_TILE)
    pltpu.sync_copy(zbuf, acca.at[pl.ds(s * ROWS_PER_TILE, ROWS_PER_TILE), :])
    pltpu.sync_copy(zbuf, accb.at[pl.ds(s * ROWS_PER_TILE, ROWS_PER_TILE), :])
    pltpu.sync_copy(mv_hbm, mv_v)
    idx16 = lax.iota(jnp.int32, 16)
    onerow = jnp.where(idx16 == 0, 1.0, 0.0).astype(jnp.float32)

    def initones(i, _):
        ones_buf[i, :] = onerow
        return 0

    lax.fori_loop(0, C, initones, 0)
    mv = mv_v[...]
    plsc.subcore_barrier()

    base0 = wid * PER_TILE

    def chunk(j, _):
        base = base0 + j * C
        pltpu.sync_copy(row_hbm.at[pl.ds(base, C)], row_v)
        pltpu.sync_copy(col_hbm.at[pl.ds(base, C)], col_v)
        cp1 = pltpu.async_copy(sfwd_hbm.at[row_v], g1, gsem)
        cp2 = pltpu.async_copy(srev_hbm.at[col_v], g2, gsem)
        cp1.wait()
        cp2.wait()

        def edge(e, _):
            z = g1[e, :] + g2[e, :]
            zl = jnp.maximum(z, NEG * z)
            a_buf[e, :] = jnp.exp(zl - mv)
            return 0

        lax.fori_loop(0, C, edge, 0)
        pltpu.sync_copy(a_buf, a_out.at[pl.ds(base, C), :])
        pltpu.sync_copy(a_buf, acca.at[row_v], add=True)
        pltpu.sync_copy(ones_buf, accb.at[col_v], add=True)
        return 0

    lax.fori_loop(0, CHUNKS, chunk, 0)
    plsc.subcore_barrier()
    sl = pl.ds(s * ROWS_PER_TILE, ROWS_PER_TILE)
    pltpu.sync_copy(acca.at[sl, :], acca_out.at[c, sl, :])
    pltpu.sync_copy(accb.at[sl, :], accb_out.at[c, sl, :])


def _kprop_body(with_ia, gidx_hbm, sidx_hbm, a_hbm, inva_hbm, tab_hbm,
                out_p,
                gi_v, si_v, ia, a_buf, w_buf, rows,
                acc, gsem):
    c = lax.axis_index("c")
    s = lax.axis_index("s")
    wid = c * 16 + s
    _zero_rows(rows, C, DF)
    for off in range(0, ROWS_PER_TILE, C):
        n = min(C, ROWS_PER_TILE - off)
        pltpu.sync_copy(rows.at[pl.ds(0, n), :],
                        acc.at[pl.ds(s * ROWS_PER_TILE + off, n), :])
    plsc.subcore_barrier()

    base0 = wid * PER_TILE

    def chunk(j, _):
        base = base0 + j * C
        pltpu.sync_copy(gidx_hbm.at[pl.ds(base, C)], gi_v)
        pltpu.sync_copy(sidx_hbm.at[pl.ds(base, C)], si_v)
        cp1 = pltpu.async_copy(tab_hbm.at[gi_v], rows, gsem)
        cp2 = pltpu.async_copy(inva_hbm.at[gi_v], ia, gsem) if with_ia else None
        pltpu.sync_copy(a_hbm.at[pl.ds(base, C), :], a_buf)
        cp1.wait()
        if with_ia:
            cp2.wait()

            def wcomp(e, _):
                w_buf[e, :] = a_buf[e, :] * ia[e, :]
                return 0

            lax.fori_loop(0, C, wcomp, 0)
        wsrc = w_buf if with_ia else a_buf

        def edge(e, _):
            wrow = wsrc[e, :]
            for hh in range(H):
                sl = pl.ds(hh * HD, HD)
                wv = jnp.full((HD,), wrow[hh], jnp.float32)
                rows[e, sl] = rows[e, sl] * wv
            return 0

        lax.fori_loop(0, C, edge, 0)
        pltpu.sync_copy(rows, acc.at[si_v], add=True)
        return 0

    lax.fori_loop(0, CHUNKS, chunk, 0)
    plsc.subcore_barrier()
    sl = pl.ds(s * ROWS_PER_TILE, ROWS_PER_TILE)
    pltpu.sync_copy(acc.at[sl, :], out_p.at[c, sl, :])


def _kthin_body(gidx_hbm, sidx_hbm, tab_hbm,
                out_p,
                gi_v, si_v, g, zbuf,
                acc, gsem):
    c = lax.axis_index("c")
    s = lax.axis_index("s")
    wid = c * 16 + s
    _zero_rows(zbuf, ROWS_PER_TILE)
    pltpu.sync_copy(zbuf, acc.at[pl.ds(s * ROWS_PER_TILE, ROWS_PER_TILE), :])
    plsc.subcore_barrier()

    base0 = wid * PER_TILE

    def chunk(j, _):
        base = base0 + j * C
        pltpu.sync_copy(gidx_hbm.at[pl.ds(base, C)], gi_v)
        pltpu.sync_copy(sidx_hbm.at[pl.ds(base, C)], si_v)
        pltpu.async_copy(tab_hbm.at[gi_v], g, gsem).wait()
        pltpu.sync_copy(g, acc.at[si_v], add=True)
        return 0

    lax.fori_loop(0, CHUNKS, chunk, 0)
    plsc.subcore_barrier()
    sl = pl.ds(s * ROWS_PER_TILE, ROWS_PER_TILE)
    pltpu.sync_copy(acc.at[sl, :], out_p.at[c, sl, :])


def kernel(x, edge_index, W1, att1, b1, W2, b2):
    f32 = jnp.float32
    x_p = jnp.zeros((NP, DF), f32).at[:N].set(x)
    pad = jnp.full((EPAD - E,), N, jnp.int32)
    row_p = jnp.concatenate([edge_index[0], pad])
    col_p = jnp.concatenate([edge_index[1], pad])
    af = att1[0, :, :HD]
    ar = att1[0, :, HD:]
    eye = jnp.eye(H, dtype=f32)
    afwd = (af[:, :, None] * eye[:, None, :]).reshape(DF, H)
    arev = (ar[:, :, None] * eye[:, None, :]).reshape(DF, H)
    afwd = jnp.pad(afwd, ((0, 0), (0, 8)))
    arev = jnp.pad(arev, ((0, 0), (0, 8)))
    w2p = jnp.pad(W2, ((0, 0), (0, 16 - NCLS)))
    b1r = b1.reshape(1, DF)
    b2p = jnp.pad(b2, (0, 16 - NCLS)).reshape(1, 16)

    xh_p, sfwd, srev, m8 = pl.pallas_call(
        _k1_body,
        out_shape=[_js(NP, DF), _js(NP, 16), _js(NP, 16), _js(8, 16)],
    )(x_p, W1, afwd, arev)
    mvec = m8[0]

    k2 = functools.partial(
        pl.kernel,
        mesh=_sc_mesh,
        compiler_params=_sc_params,
        out_type=[_js(EPAD, 16), _js(2, NP, 16), _js(2, NP, 16)],
        scratch_types=[
            pltpu.VMEM((C,), jnp.int32),
            pltpu.VMEM((C,), jnp.int32),
            pltpu.VMEM((C, 16), f32),
            pltpu.VMEM((C, 16), f32),
            pltpu.VMEM((C, 16), f32),
            pltpu.VMEM((C, 16), f32),
            pltpu.VMEM((ROWS_PER_TILE, 16), f32),
            pltpu.VMEM((16,), f32),
            pltpu.VMEM_SHARED((NP, 16), f32),
            pltpu.VMEM_SHARED((NP, 16), f32),
            pltpu.SemaphoreType.DMA,
        ],
    )(_k2_body)
    a_e, acca_p, accb_p = k2(row_p, col_p, sfwd, srev, mvec)

    inva, dinvw, binvw = pl.pallas_call(
        _k3_body,
        grid=(NP // K3_BR,),
        in_specs=[
            pl.BlockSpec((2, K3_BR, 16), lambda i: (0, i, 0)),
            pl.BlockSpec((2, K3_BR, 16), lambda i: (0, i, 0)),
        ],
        out_specs=[
            pl.BlockSpec((K3_BR, 16), lambda i: (i, 0)),
            pl.BlockSpec((K3_BR, DF), lambda i: (i, 0)),
            pl.BlockSpec((K3_BR, DF), lambda i: (i, 0)),
        ],
        out_shape=[_js(NP, 16), _js(NP, DF), _js(NP, DF)],
    )(acca_p, accb_p)

    def _mk_kprop(with_ia):
        return functools.partial(
            pl.kernel,
            mesh=_sc_mesh,
            compiler_params=_sc_params,
            out_type=[_js(2, NP, DF)],
            scratch_types=[
                pltpu.VMEM((C,), jnp.int32),
                pltpu.VMEM((C,), jnp.int32),
                pltpu.VMEM((C, 16), f32),
                pltpu.VMEM((C, 16), f32),
                pltpu.VMEM((C, 16), f32),
                pltpu.VMEM((C, DF), f32),
                pltpu.VMEM_SHARED((NP, DF), f32),
                pltpu.SemaphoreType.DMA,
            ],
        )(functools.partial(_kprop_body, with_ia))

    (oute_p,) = _mk_kprop(True)(row_p, col_p, a_e, inva, xh_p)

    oute_b = pl.pallas_call(_k5_body, out_shape=_js(NP, DF))(oute_p, binvw)

    (out1_p,) = _mk_kprop(False)(col_p, row_p, a_e, inva, oute_b)

    x2p = pl.pallas_call(_k7_body, out_shape=_js(NP, 16))(out1_p, dinvw, b1r, w2p)

    kthin = functools.partial(
        pl.kernel,
        mesh=_sc_mesh,
        compiler_params=_sc_params,
        out_type=[_js(2, NP, 16)],
        scratch_types=[
            pltpu.VMEM((C,), jnp.int32),
            pltpu.VMEM((C,), jnp.int32),
            pltpu.VMEM((C, 16), f32),
            pltpu.VMEM((ROWS_PER_TILE, 16), f32),
            pltpu.VMEM_SHARED((NP, 16), f32),
            pltpu.SemaphoreType.DMA,
        ],
    )(_kthin_body)
    (acc3_p,) = kthin(row_p, col_p, x2p)

    oute2_b = pl.pallas_call(_k9_body, out_shape=_js(NP, 16))(acc3_p, inva)

    (acc4_p,) = kthin(col_p, row_p, oute2_b)

    logp = pl.pallas_call(_k11_body, out_shape=_js(NP, 16))(acc4_p, inva, b2p)
    return logp[:N, :NCLS]


# speedup vs baseline: 1.6725x; 1.2290x over previous
import functools

import jax
import jax.numpy as jnp
from jax import lax
from jax.experimental import pallas as pl
from jax.experimental.pallas import tpu as pltpu
from jax.experimental.pallas import tpu_sc as plsc

N = 10000
E = 320000
DF = 128
H = 8
HD = 16
NCLS = 7
NEG = 0.2

NP = 10240
C = 128
NTILES = 32
CHUNKS = E // (NTILES * C) + 1
PER_TILE = CHUNKS * C
EPAD = PER_TILE * NTILES
ROWS_PER_TILE = NP // 16

_sc_mesh = plsc.VectorSubcoreMesh(core_axis_name="c", subcore_axis_name="s")
_sc_params = pltpu.CompilerParams(use_tc_tiling_on_sc=False)


def _js(*shape):
    return jax.ShapeDtypeStruct(shape, jnp.float32)


def _k1_body(x_ref, w1_ref, afwd_ref, arev_ref, xh_out, sfwd_out, srev_out, m_out):
    x = x_ref[...]
    xh = jnp.dot(x, w1_ref[...], preferred_element_type=jnp.float32)
    xh_out[...] = xh
    sf = jnp.dot(xh, afwd_ref[...], preferred_element_type=jnp.float32)
    sr = jnp.dot(xh, arev_ref[...], preferred_element_type=jnp.float32)
    sfwd_out[...] = sf
    srev_out[...] = sr
    msum = jnp.max(sf, axis=0) + jnp.max(sr, axis=0)
    m16 = jnp.maximum(msum, NEG * msum)
    m_out[...] = jnp.broadcast_to(m16[None, :], (8, 16))


K3_BR = 2048


def _k3_body(acca_ref, accb_ref, inva_out, dinvw_out, binvw_out):
    br = K3_BR
    pa = acca_ref[0] + acca_ref[1]
    pb = accb_ref[0] + accb_ref[1]
    degn = pa[:, 8:9]
    dege = pb[:, 0:1]
    dinv = jnp.where(degn > 0, 1.0 / degn, 0.0)
    binv = jnp.where(dege > 0, 1.0 / dege, 0.0)
    ia = 1.0 / (pa + 1e-16)
    colid = lax.broadcasted_iota(jnp.int32, (br, 16), 1)
    out = jnp.where(colid < 8, ia, 0.0)
    out = jnp.where(colid == 8, jnp.broadcast_to(dinv, (br, 16)), out)
    out = jnp.where(colid == 9, jnp.broadcast_to(binv, (br, 16)), out)
    inva_out[...] = out
    ia_rep = jnp.broadcast_to(ia[:, :8, None], (br, 8, HD)).reshape(br, DF)
    dinvw_out[...] = jnp.broadcast_to(dinv, (br, DF)) * ia_rep
    binvw_out[...] = jnp.broadcast_to(binv, (br, DF))


def _k5_body(p_ref, binvw_ref, out):
    out[...] = (p_ref[0] + p_ref[1]) * binvw_ref[...]


def _k7_body(p_ref, dinvw_ref, b1_ref, w2p_ref, out):
    o = (p_ref[0] + p_ref[1]) * dinvw_ref[...] + b1_ref[...]
    h = jnp.where(o > 0, o, jnp.exp(o) - 1.0)
    out[...] = jnp.dot(h, w2p_ref[...], preferred_element_type=jnp.float32)


def _k9_body(p_ref, inva_ref, out):
    binv = inva_ref[:, 9:10]
    out[...] = (p_ref[0] + p_ref[1]) * jnp.broadcast_to(binv, (NP, 16))


def _k11_body(p_ref, inva_ref, b2p_ref, out):
    dinv = inva_ref[:, 8:9]
    o = (p_ref[0] + p_ref[1]) * jnp.broadcast_to(dinv, (NP, 16)) + b2p_ref[...]
    colid = lax.broadcasted_iota(jnp.int32, (NP, 16), 1)
    o = jnp.where(colid < NCLS, o, -1e30)
    m = jnp.max(o, axis=1, keepdims=True)
    ex = jnp.where(colid < NCLS, jnp.exp(o - m), 0.0)
    lse = jnp.log(jnp.sum(ex, axis=1, keepdims=True))
    out[...] = o - m - lse


def _zero_rows(buf, nrows, width=16):
    z = jnp.zeros((16,), jnp.float32)

    def body(i, _):
        for k in range(width // 16):
            buf[i, pl.ds(k * 16, 16)] = z
        return 0

    lax.fori_loop(0, nrows, body, 0)


def _k2_body(row_hbm, col_hbm, sfwd_hbm, srev_hbm, mv_hbm,
             a_out, acca_out, accb_out,
             row_v, col_v, g1, g2, a_buf, ones_buf, zbuf, mv_v,
             acca, accb, gsem):
    c = lax.axis_index("c")
    s = lax.axis_index("s")
    wid = c * 16 + s
    _zero_rows(zbuf, ROWS_PER_TILE)
    pltpu.sync_copy(zbuf, acca.at[pl.ds(s * ROWS_PER_TILE, ROWS_PER_TILE), :])
    pltpu.sync_copy(zbuf, accb.at[pl.ds(s * ROWS_PER_TILE, ROWS_PER_TILE), :])
    pltpu.sync_copy(mv_hbm, mv_v)
    idx16 = lax.iota(jnp.int32, 16)
    onerow = jnp.where(idx16 == 0, 1.0, 0.0).astype(jnp.float32)

    def initones(i, _):
        ones_buf[i, :] = onerow
        return 0

    lax.fori_loop(0, C, initones, 0)
    mv = mv_v[...]
    pltpu.sync_copy(row_hbm.at[pl.ds(wid * CHUNKS, CHUNKS), :], row_v)
    pltpu.sync_copy(col_hbm.at[pl.ds(wid * CHUNKS, CHUNKS), :], col_v)
    plsc.subcore_barrier()

    base0 = wid * PER_TILE

    def chunk(j, _):
        base = base0 + j * C
        cp1 = pltpu.async_copy(sfwd_hbm.at[row_v.at[j]], g1, gsem)
        cp2 = pltpu.async_copy(srev_hbm.at[col_v.at[j]], g2, gsem)
        cp1.wait()
        cp2.wait()

        def edge(e, _):
            z = g1[e, :] + g2[e, :]
            zl = jnp.maximum(z, NEG * z)
            a_buf[e, :] = jnp.exp(zl - mv)
            return 0

        lax.fori_loop(0, C, edge, 0)
        pltpu.sync_copy(a_buf, a_out.at[pl.ds(base, C), :])
        pltpu.sync_copy(a_buf, acca.at[row_v.at[j]], add=True)
        pltpu.sync_copy(ones_buf, accb.at[col_v.at[j]], add=True)
        return 0

    lax.fori_loop(0, CHUNKS, chunk, 0)
    plsc.subcore_barrier()
    sl = pl.ds(s * ROWS_PER_TILE, ROWS_PER_TILE)
    pltpu.sync_copy(acca.at[sl, :], acca_out.at[c, sl, :])
    pltpu.sync_copy(accb.at[sl, :], accb_out.at[c, sl, :])


def _kprop_body(with_ia, gidx_hbm, sidx_hbm, a_hbm, inva_hbm, tab_hbm,
                out_p,
                gi_v, si_v, ia, a_buf, w_buf, rows,
                acc, gsem):
    c = lax.axis_index("c")
    s = lax.axis_index("s")
    wid = c * 16 + s
    _zero_rows(rows, C, DF)
    for off in range(0, ROWS_PER_TILE, C):
        n = min(C, ROWS_PER_TILE - off)
        pltpu.sync_copy(rows.at[pl.ds(0, n), :],
                        acc.at[pl.ds(s * ROWS_PER_TILE + off, n), :])
    plsc.subcore_barrier()

    pltpu.sync_copy(gidx_hbm.at[pl.ds(wid * CHUNKS, CHUNKS), :], gi_v)
    pltpu.sync_copy(sidx_hbm.at[pl.ds(wid * CHUNKS, CHUNKS), :], si_v)
    base0 = wid * PER_TILE

    def chunk(j, _):
        base = base0 + j * C
        cp1 = pltpu.async_copy(tab_hbm.at[gi_v.at[j]], rows, gsem)
        cp2 = (pltpu.async_copy(inva_hbm.at[gi_v.at[j]], ia, gsem)
               if with_ia else None)
        pltpu.sync_copy(a_hbm.at[pl.ds(base, C), :], a_buf)
        cp1.wait()
        if with_ia:
            cp2.wait()

            def wcomp(e, _):
                w_buf[e, :] = a_buf[e, :] * ia[e, :]
                return 0

            lax.fori_loop(0, C, wcomp, 0)
        wsrc = w_buf if with_ia else a_buf

        def edge(e, _):
            wrow = wsrc[e, :]
            for hh in range(H):
                sl = pl.ds(hh * HD, HD)
                wv = jnp.full((HD,), wrow[hh], jnp.float32)
                rows[e, sl] = rows[e, sl] * wv
            return 0

        lax.fori_loop(0, C, edge, 0)
        pltpu.sync_copy(rows, acc.at[si_v.at[j]], add=True)
        return 0

    lax.fori_loop(0, CHUNKS, chunk, 0)
    plsc.subcore_barrier()
    sl = pl.ds(s * ROWS_PER_TILE, ROWS_PER_TILE)
    pltpu.sync_copy(acc.at[sl, :], out_p.at[c, sl, :])


def _kthin_body(gidx_hbm, sidx_hbm, tab_hbm,
                out_p,
                gi_v, si_v, g, zbuf,
                acc, gsem):
    c = lax.axis_index("c")
    s = lax.axis_index("s")
    wid = c * 16 + s
    _zero_rows(zbuf, ROWS_PER_TILE)
    pltpu.sync_copy(zbuf, acc.at[pl.ds(s * ROWS_PER_TILE, ROWS_PER_TILE), :])
    plsc.subcore_barrier()

    pltpu.sync_copy(gidx_hbm.at[pl.ds(wid * CHUNKS, CHUNKS), :], gi_v)
    pltpu.sync_copy(sidx_hbm.at[pl.ds(wid * CHUNKS, CHUNKS), :], si_v)
    base0 = wid * PER_TILE

    def chunk(j, _):
        pltpu.async_copy(tab_hbm.at[gi_v.at[j]], g, gsem).wait()
        pltpu.sync_copy(g, acc.at[si_v.at[j]], add=True)
        return 0

    lax.fori_loop(0, CHUNKS, chunk, 0)
    plsc.subcore_barrier()
    sl = pl.ds(s * ROWS_PER_TILE, ROWS_PER_TILE)
    pltpu.sync_copy(acc.at[sl, :], out_p.at[c, sl, :])


def kernel(x, edge_index, W1, att1, b1, W2, b2):
    f32 = jnp.float32
    x_p = jnp.zeros((NP, DF), f32).at[:N].set(x)
    pad = jnp.full((EPAD - E,), N, jnp.int32)
    row_p = jnp.concatenate([edge_index[0], pad]).reshape(EPAD // C, C)
    col_p = jnp.concatenate([edge_index[1], pad]).reshape(EPAD // C, C)
    af = att1[0, :, :HD]
    ar = att1[0, :, HD:]
    eye = jnp.eye(H, dtype=f32)
    afwd = (af[:, :, None] * eye[:, None, :]).reshape(DF, H)
    arev = (ar[:, :, None] * eye[:, None, :]).reshape(DF, H)
    afwd = jnp.pad(afwd, ((0, 0), (0, 8)))
    arev = jnp.pad(arev, ((0, 0), (0, 8)))
    w2p = jnp.pad(W2, ((0, 0), (0, 16 - NCLS)))
    b1r = b1.reshape(1, DF)
    b2p = jnp.pad(b2, (0, 16 - NCLS)).reshape(1, 16)

    xh_p, sfwd, srev, m8 = pl.pallas_call(
        _k1_body,
        out_shape=[_js(NP, DF), _js(NP, 16), _js(NP, 16), _js(8, 16)],
    )(x_p, W1, afwd, arev)
    mvec = m8[0]

    k2 = functools.partial(
        pl.kernel,
        mesh=_sc_mesh,
        compiler_params=_sc_params,
        out_type=[_js(EPAD, 16), _js(2, NP, 16), _js(2, NP, 16)],
        scratch_types=[
            pltpu.VMEM((CHUNKS, C), jnp.int32),
            pltpu.VMEM((CHUNKS, C), jnp.int32),
            pltpu.VMEM((C, 16), f32),
            pltpu.VMEM((C, 16), f32),
            pltpu.VMEM((C, 16), f32),
            pltpu.VMEM((C, 16), f32),
            pltpu.VMEM((ROWS_PER_TILE, 16), f32),
            pltpu.VMEM((16,), f32),
            pltpu.VMEM_SHARED((NP, 16), f32),
            pltpu.VMEM_SHARED((NP, 16), f32),
            pltpu.SemaphoreType.DMA,
        ],
    )(_k2_body)
    a_e, acca_p, accb_p = k2(row_p, col_p, sfwd, srev, mvec)

    inva, dinvw, binvw = pl.pallas_call(
        _k3_body,
        grid=(NP // K3_BR,),
        in_specs=[
            pl.BlockSpec((2, K3_BR, 16), lambda i: (0, i, 0)),
            pl.BlockSpec((2, K3_BR, 16), lambda i: (0, i, 0)),
        ],
        out_specs=[
            pl.BlockSpec((K3_BR, 16), lambda i: (i, 0)),
            pl.BlockSpec((K3_BR, DF), lambda i: (i, 0)),
            pl.BlockSpec((K3_BR, DF), lambda i: (i, 0)),
        ],
        out_shape=[_js(NP, 16), _js(NP, DF), _js(NP, DF)],
    )(acca_p, accb_p)

    def _mk_kprop(with_ia):
        return functools.partial(
            pl.kernel,
            mesh=_sc_mesh,
            compiler_params=_sc_params,
            out_type=[_js(2, NP, DF)],
            scratch_types=[
                pltpu.VMEM((CHUNKS, C), jnp.int32),
                pltpu.VMEM((CHUNKS, C), jnp.int32),
                pltpu.VMEM((C, 16), f32),
                pltpu.VMEM((C, 16), f32),
                pltpu.VMEM((C, 16), f32),
                pltpu.VMEM((C, DF), f32),
                pltpu.VMEM_SHARED((NP, DF), f32),
                pltpu.SemaphoreType.DMA,
            ],
        )(functools.partial(_kprop_body, with_ia))

    (oute_p,) = _mk_kprop(True)(row_p, col_p, a_e, inva, xh_p)

    oute_b = pl.pallas_call(_k5_body, out_shape=_js(NP, DF))(oute_p, binvw)

    (out1_p,) = _mk_kprop(False)(col_p, row_p, a_e, inva, oute_b)

    x2p = pl.pallas_call(_k7_body, out_shape=_js(NP, 16))(out1_p, dinvw, b1r, w2p)

    kthin = functools.partial(
        pl.kernel,
        mesh=_sc_mesh,
        compiler_params=_sc_params,
        out_type=[_js(2, NP, 16)],
        scratch_types=[
            pltpu.VMEM((CHUNKS, C), jnp.int32),
            pltpu.VMEM((CHUNKS, C), jnp.int32),
            pltpu.VMEM((C, 16), f32),
            pltpu.VMEM((ROWS_PER_TILE, 16), f32),
            pltpu.VMEM_SHARED((NP, 16), f32),
            pltpu.SemaphoreType.DMA,
        ],
    )(_kthin_body)
    (acc3_p,) = kthin(row_p, col_p, x2p)

    oute2_b = pl.pallas_call(_k9_body, out_shape=_js(NP, 16))(acc3_p, inva)

    (acc4_p,) = kthin(col_p, row_p, oute2_b)

    logp = pl.pallas_call(_k11_body, out_shape=_js(NP, 16))(acc4_p, inva, b2p)
    return logp[:N, :NCLS]


# speedup vs baseline: 1.6733x; 1.0005x over previous
import functools

import jax
import jax.numpy as jnp
from jax import lax
from jax.experimental import pallas as pl
from jax.experimental.pallas import tpu as pltpu
from jax.experimental.pallas import tpu_sc as plsc

N = 10000
E = 320000
DF = 128
H = 8
HD = 16
NCLS = 7
NEG = 0.2

NP = 10240
C = 128
NTILES = 32
CHUNKS = E // (NTILES * C) + 1
PER_TILE = CHUNKS * C
EPAD = PER_TILE * NTILES
ROWS_PER_TILE = NP // 16

_sc_mesh = plsc.VectorSubcoreMesh(core_axis_name="c", subcore_axis_name="s")
_sc_params = pltpu.CompilerParams(use_tc_tiling_on_sc=False)


def _js(*shape):
    return jax.ShapeDtypeStruct(shape, jnp.float32)


def _k1_body(x_ref, w1_ref, afwd_ref, arev_ref, xh_out, sfwd_out, srev_out, m_out):
    x = x_ref[...]
    xh = jnp.dot(x, w1_ref[...], preferred_element_type=jnp.float32)
    xh_out[...] = xh
    sf = jnp.dot(xh, afwd_ref[...], preferred_element_type=jnp.float32)
    sr = jnp.dot(xh, arev_ref[...], preferred_element_type=jnp.float32)
    sfwd_out[...] = sf
    srev_out[...] = sr
    msum = jnp.max(sf, axis=0) + jnp.max(sr, axis=0)
    m16 = jnp.maximum(msum, NEG * msum)
    m_out[...] = jnp.broadcast_to(m16[None, :], (8, 16))


K3_BR = 2048


def _k3_body(acca_ref, accb_ref, inva_out, dinvw_out, binvw_out):
    br = K3_BR
    pa = acca_ref[0] + acca_ref[1]
    pb = accb_ref[0] + accb_ref[1]
    degn = pa[:, 8:9]
    dege = pb[:, 0:1]
    dinv = jnp.where(degn > 0, 1.0 / degn, 0.0)
    binv = jnp.where(dege > 0, 1.0 / dege, 0.0)
    ia = 1.0 / (pa + 1e-16)
    colid = lax.broadcasted_iota(jnp.int32, (br, 16), 1)
    out = jnp.where(colid < 8, ia, 0.0)
    out = jnp.where(colid == 8, jnp.broadcast_to(dinv, (br, 16)), out)
    out = jnp.where(colid == 9, jnp.broadcast_to(binv, (br, 16)), out)
    inva_out[...] = out
    ia_rep = jnp.broadcast_to(ia[:, :8, None], (br, 8, HD)).reshape(br, DF)
    dinvw_out[...] = jnp.broadcast_to(dinv, (br, DF)) * ia_rep
    binvw_out[...] = jnp.broadcast_to(binv, (br, DF))


def _k5_body(p_ref, binvw_ref, out):
    out[...] = (p_ref[0] + p_ref[1]) * binvw_ref[...]


def _k7_body(p_ref, dinvw_ref, b1_ref, w2p_ref, out):
    o = (p_ref[0] + p_ref[1]) * dinvw_ref[...] + b1_ref[...]
    h = jnp.where(o > 0, o, jnp.exp(o) - 1.0)
    out[...] = jnp.dot(h, w2p_ref[...], preferred_element_type=jnp.float32)


def _k9_body(p_ref, inva_ref, out):
    binv = inva_ref[:, 9:10]
    out[...] = (p_ref[0] + p_ref[1]) * jnp.broadcast_to(binv, (NP, 16))


def _k11_body(p_ref, inva_ref, b2p_ref, out):
    dinv = inva_ref[:, 8:9]
    o = (p_ref[0] + p_ref[1]) * jnp.broadcast_to(dinv, (NP, 16)) + b2p_ref[...]
    colid = lax.broadcasted_iota(jnp.int32, (NP, 16), 1)
    o = jnp.where(colid < NCLS, o, -1e30)
    m = jnp.max(o, axis=1, keepdims=True)
    ex = jnp.where(colid < NCLS, jnp.exp(o - m), 0.0)
    lse = jnp.log(jnp.sum(ex, axis=1, keepdims=True))
    out[...] = o - m - lse


def _zero_rows(buf, nrows, width=16):
    z = jnp.zeros((16,), jnp.float32)

    def body(i, _):
        for k in range(width // 16):
            buf[i, pl.ds(k * 16, 16)] = z
        return 0

    lax.fori_loop(0, nrows, body, 0)


def _k2_body(row_hbm, col_hbm, sfwd_hbm, srev_hbm, mv_hbm,
             a_out, acca_out, accb_out,
             row_v, col_v, g1, g2, a_buf, ones_buf, zbuf, mv_v,
             acca, accb, gsem):
    c = lax.axis_index("c")
    s = lax.axis_index("s")
    wid = c * 16 + s
    _zero_rows(zbuf, ROWS_PER_TILE)
    pltpu.sync_copy(zbuf, acca.at[pl.ds(s * ROWS_PER_TILE, ROWS_PER_TILE), :])
    pltpu.sync_copy(zbuf, accb.at[pl.ds(s * ROWS_PER_TILE, ROWS_PER_TILE), :])
    pltpu.sync_copy(mv_hbm, mv_v)
    idx16 = lax.iota(jnp.int32, 16)
    onerow = jnp.where(idx16 == 0, 1.0, 0.0).astype(jnp.float32)

    def initones(i, _):
        ones_buf[i, :] = onerow
        return 0

    lax.fori_loop(0, C, initones, 0)
    mv = mv_v[...]
    pltpu.sync_copy(row_hbm.at[pl.ds(wid * CHUNKS, CHUNKS), :], row_v)
    pltpu.sync_copy(col_hbm.at[pl.ds(wid * CHUNKS, CHUNKS), :], col_v)
    plsc.subcore_barrier()

    base0 = wid * PER_TILE

    def chunk(j, _):
        base = base0 + j * C
        cp1 = pltpu.async_copy(sfwd_hbm.at[row_v.at[j]], g1, gsem)
        cp2 = pltpu.async_copy(srev_hbm.at[col_v.at[j]], g2, gsem)
        cp1.wait()
        cp2.wait()

        def edge(e, _):
            z = g1[e, :] + g2[e, :]
            zl = jnp.maximum(z, NEG * z)
            a_buf[e, :] = jnp.exp(zl - mv)
            return 0

        lax.fori_loop(0, C, edge, 0)
        pltpu.sync_copy(a_buf, a_out.at[pl.ds(base, C), :])
        pltpu.sync_copy(a_buf, acca.at[row_v.at[j]], add=True)
        pltpu.sync_copy(ones_buf, accb.at[col_v.at[j]], add=True)
        return 0

    lax.fori_loop(0, CHUNKS, chunk, 0)
    plsc.subcore_barrier()
    sl = pl.ds(s * ROWS_PER_TILE, ROWS_PER_TILE)
    pltpu.sync_copy(acca.at[sl, :], acca_out.at[c, sl, :])
    pltpu.sync_copy(accb.at[sl, :], accb_out.at[c, sl, :])


def _kprop_body(with_ia, gidx_hbm, sidx_hbm, a_hbm, inva_hbm, tab_hbm,
                out_p,
                gi_v, si_v, ia, a_buf, w_buf, rows,
                acc, gsem, asem):
    c = lax.axis_index("c")
    s = lax.axis_index("s")
    wid = c * 16 + s
    _zero_rows(rows, C, DF)
    for off in range(0, ROWS_PER_TILE, C):
        n = min(C, ROWS_PER_TILE - off)
        pltpu.sync_copy(rows.at[pl.ds(0, n), :],
                        acc.at[pl.ds(s * ROWS_PER_TILE + off, n), :])
    plsc.subcore_barrier()

    pltpu.sync_copy(gidx_hbm.at[pl.ds(wid * CHUNKS, CHUNKS), :], gi_v)
    pltpu.sync_copy(sidx_hbm.at[pl.ds(wid * CHUNKS, CHUNKS), :], si_v)
    base0 = wid * PER_TILE

    def chunk(j, _):
        base = base0 + j * C
        cp1 = pltpu.async_copy(tab_hbm.at[gi_v.at[j]], rows, gsem)
        cp2 = (pltpu.async_copy(inva_hbm.at[gi_v.at[j]], ia, gsem)
               if with_ia else None)
        cp3 = pltpu.async_copy(a_hbm.at[pl.ds(base, C), :], a_buf, asem)
        cp1.wait()
        cp3.wait()
        if with_ia:
            cp2.wait()

            def wcomp(e, _):
                w_buf[e, :] = a_buf[e, :] * ia[e, :]
                return 0

            lax.fori_loop(0, C, wcomp, 0)
        wsrc = w_buf if with_ia else a_buf

        def edge(e, _):
            wrow = wsrc[e, :]
            for hh in range(H):
                sl = pl.ds(hh * HD, HD)
                wv = jnp.full((HD,), wrow[hh], jnp.float32)
                rows[e, sl] = rows[e, sl] * wv
            return 0

        lax.fori_loop(0, C, edge, 0)
        pltpu.sync_copy(rows, acc.at[si_v.at[j]], add=True)
        return 0

    lax.fori_loop(0, CHUNKS, chunk, 0)
    plsc.subcore_barrier()
    sl = pl.ds(s * ROWS_PER_TILE, ROWS_PER_TILE)
    pltpu.sync_copy(acc.at[sl, :], out_p.at[c, sl, :])


def _kthin_body(gidx_hbm, sidx_hbm, tab_hbm,
                out_p,
                gi_v, si_v, g, zbuf,
                acc, gsem):
    c = lax.axis_index("c")
    s = lax.axis_index("s")
    wid = c * 16 + s
    _zero_rows(zbuf, ROWS_PER_TILE)
    pltpu.sync_copy(zbuf, acc.at[pl.ds(s * ROWS_PER_TILE, ROWS_PER_TILE), :])
    plsc.subcore_barrier()

    pltpu.sync_copy(gidx_hbm.at[pl.ds(wid * CHUNKS, CHUNKS), :], gi_v)
    pltpu.sync_copy(sidx_hbm.at[pl.ds(wid * CHUNKS, CHUNKS), :], si_v)
    base0 = wid * PER_TILE

    def chunk(j, _):
        pltpu.async_copy(tab_hbm.at[gi_v.at[j]], g, gsem).wait()
        pltpu.sync_copy(g, acc.at[si_v.at[j]], add=True)
        return 0

    lax.fori_loop(0, CHUNKS, chunk, 0)
    plsc.subcore_barrier()
    sl = pl.ds(s * ROWS_PER_TILE, ROWS_PER_TILE)
    pltpu.sync_copy(acc.at[sl, :], out_p.at[c, sl, :])


def kernel(x, edge_index, W1, att1, b1, W2, b2):
    f32 = jnp.float32
    x_p = jnp.zeros((NP, DF), f32).at[:N].set(x)
    pad = jnp.full((EPAD - E,), N, jnp.int32)
    row_p = jnp.concatenate([edge_index[0], pad]).reshape(EPAD // C, C)
    col_p = jnp.concatenate([edge_index[1], pad]).reshape(EPAD // C, C)
    af = att1[0, :, :HD]
    ar = att1[0, :, HD:]
    eye = jnp.eye(H, dtype=f32)
    afwd = (af[:, :, None] * eye[:, None, :]).reshape(DF, H)
    arev = (ar[:, :, None] * eye[:, None, :]).reshape(DF, H)
    afwd = jnp.pad(afwd, ((0, 0), (0, 8)))
    arev = jnp.pad(arev, ((0, 0), (0, 8)))
    w2p = jnp.pad(W2, ((0, 0), (0, 16 - NCLS)))
    b1r = b1.reshape(1, DF)
    b2p = jnp.pad(b2, (0, 16 - NCLS)).reshape(1, 16)

    xh_p, sfwd, srev, m8 = pl.pallas_call(
        _k1_body,
        out_shape=[_js(NP, DF), _js(NP, 16), _js(NP, 16), _js(8, 16)],
    )(x_p, W1, afwd, arev)
    mvec = m8[0]

    k2 = functools.partial(
        pl.kernel,
        mesh=_sc_mesh,
        compiler_params=_sc_params,
        out_type=[_js(EPAD, 16), _js(2, NP, 16), _js(2, NP, 16)],
        scratch_types=[
            pltpu.VMEM((CHUNKS, C), jnp.int32),
            pltpu.VMEM((CHUNKS, C), jnp.int32),
            pltpu.VMEM((C, 16), f32),
            pltpu.VMEM((C, 16), f32),
            pltpu.VMEM((C, 16), f32),
            pltpu.VMEM((C, 16), f32),
            pltpu.VMEM((ROWS_PER_TILE, 16), f32),
            pltpu.VMEM((16,), f32),
            pltpu.VMEM_SHARED((NP, 16), f32),
            pltpu.VMEM_SHARED((NP, 16), f32),
            pltpu.SemaphoreType.DMA,
        ],
    )(_k2_body)
    a_e, acca_p, accb_p = k2(row_p, col_p, sfwd, srev, mvec)

    inva, dinvw, binvw = pl.pallas_call(
        _k3_body,
        grid=(NP // K3_BR,),
        in_specs=[
            pl.BlockSpec((2, K3_BR, 16), lambda i: (0, i, 0)),
            pl.BlockSpec((2, K3_BR, 16), lambda i: (0, i, 0)),
        ],
        out_specs=[
            pl.BlockSpec((K3_BR, 16), lambda i: (i, 0)),
            pl.BlockSpec((K3_BR, DF), lambda i: (i, 0)),
            pl.BlockSpec((K3_BR, DF), lambda i: (i, 0)),
        ],
        out_shape=[_js(NP, 16), _js(NP, DF), _js(NP, DF)],
    )(acca_p, accb_p)

    def _mk_kprop(with_ia):
        return functools.partial(
            pl.kernel,
            mesh=_sc_mesh,
            compiler_params=_sc_params,
            out_type=[_js(2, NP, DF)],
            scratch_types=[
                pltpu.VMEM((CHUNKS, C), jnp.int32),
                pltpu.VMEM((CHUNKS, C), jnp.int32),
                pltpu.VMEM((C, 16), f32),
                pltpu.VMEM((C, 16), f32),
                pltpu.VMEM((C, 16), f32),
                pltpu.VMEM((C, DF), f32),
                pltpu.VMEM_SHARED((NP, DF), f32),
                pltpu.SemaphoreType.DMA,
                pltpu.SemaphoreType.DMA,
            ],
        )(functools.partial(_kprop_body, with_ia))

    (oute_p,) = _mk_kprop(True)(row_p, col_p, a_e, inva, xh_p)

    oute_b = pl.pallas_call(_k5_body, out_shape=_js(NP, DF))(oute_p, binvw)

    (out1_p,) = _mk_kprop(False)(col_p, row_p, a_e, inva, oute_b)

    x2p = pl.pallas_call(_k7_body, out_shape=_js(NP, 16))(out1_p, dinvw, b1r, w2p)

    kthin = functools.partial(
        pl.kernel,
        mesh=_sc_mesh,
        compiler_params=_sc_params,
        out_type=[_js(2, NP, 16)],
        scratch_types=[
            pltpu.VMEM((CHUNKS, C), jnp.int32),
            pltpu.VMEM((CHUNKS, C), jnp.int32),
            pltpu.VMEM((C, 16), f32),
            pltpu.VMEM((ROWS_PER_TILE, 16), f32),
            pltpu.VMEM_SHARED((NP, 16), f32),
            pltpu.SemaphoreType.DMA,
        ],
    )(_kthin_body)
    (acc3_p,) = kthin(row_p, col_p, x2p)

    oute2_b = pl.pallas_call(_k9_body, out_shape=_js(NP, 16))(acc3_p, inva)

    (acc4_p,) = kthin(col_p, row_p, oute2_b)

    logp = pl.pallas_call(_k11_body, out_shape=_js(NP, 16))(acc4_p, inva, b2p)
    return logp[:N, :NCLS]
